# baseline (speedup 1.0000x reference)
import functools

import jax
import jax.numpy as jnp
from jax import lax
from jax.experimental import pallas as pl
from jax.experimental.pallas import tpu as pltpu
from jax.experimental.pallas import tpu_sc as plsc

NUM_EMBEDDINGS = 1000000
EMBED_DIM = 64
N_TOKENS = 819200

_NC = 2
_NS = 16
_NW = _NC * _NS

_CHUNK = 128
_B_PER_W = N_TOKENS // _NW
_N_CHUNKS = _B_PER_W // _CHUNK
_R = 4
_RB = EMBED_DIM // 8
_N_CB = N_TOKENS // _CHUNK

_mesh = plsc.VectorSubcoreMesh(core_axis_name="c", subcore_axis_name="s")


@functools.partial(
    pl.kernel,
    out_type=jax.ShapeDtypeStruct((_RB, _N_CB, 8, _CHUNK), jnp.float32),
    mesh=_mesh,
    scratch_types=[
        pltpu.VMEM((_B_PER_W,), jnp.int32),
        pltpu.VMEM((_R, _CHUNK, EMBED_DIM), jnp.float32),
        pltpu.VMEM((_R, _RB, 8, _CHUNK), jnp.float32),
        pltpu.SemaphoreType.DMA,
        pltpu.SemaphoreType.DMA,
    ],
    compiler_params=pltpu.CompilerParams(
        use_tc_tiling_on_sc=False, needs_layout_passes=False),
)
def _gather_kernel(tokens_hbm, table_hbm, out_hbm, idx_v, rows_v, tiles_v,
                   gsem, wsem):
    wid = lax.axis_index("s") * _NC + lax.axis_index("c")
    base = wid * _B_PER_W
    cb0 = wid * _N_CHUNKS
    pltpu.sync_copy(tokens_hbm.at[pl.ds(base, _B_PER_W)], idx_v)

    lane = lax.iota(jnp.int32, 16)
    ln_idx = [lane + 16 * g for g in range(_CHUNK // 16)]
    col_idx = [jnp.full((16,), c, jnp.int32) for c in range(EMBED_DIM)]

    def fire_gather(c):
        pltpu.async_copy(
            table_hbm.at[idx_v.at[pl.ds(c * _CHUNK, _CHUNK)]],
            rows_v.at[c % _R], gsem)

    def drain_gather(c):
        pltpu.make_async_copy(
            table_hbm.at[idx_v.at[pl.ds(c * _CHUNK, _CHUNK)]],
            rows_v.at[c % _R], gsem).wait()

    def transpose_chunk(c):
        slot = c % _R
        rows = rows_v.at[slot]
        for rb in range(_RB):
            for sl in range(8):
                col = rb * 8 + sl
                for g in range(_CHUNK // 16):
                    v = plsc.load_gather(rows, [ln_idx[g], col_idx[col]])
                    tiles_v[slot, rb, sl, pl.ds(16 * g, 16)] = v

    def fire_writes(c):
        for rb in range(_RB):
            pltpu.async_copy(
                tiles_v.at[c % _R, rb], out_hbm.at[rb, cb0 + c], wsem)

    def drain_writes(c):
        for rb in range(_RB):
            pltpu.make_async_copy(
                tiles_v.at[c % _R, rb], out_hbm.at[rb, cb0 + c], wsem).wait()

    for c in range(3):
        fire_gather(c)

    def warm_body(c, carry):
        drain_gather(c)
        fire_gather(c + 3)
        transpose_chunk(c)
        fire_writes(c)
        return carry

    def steady_body(c, carry):
        drain_gather(c)
        drain_writes(c - _R)
        fire_gather(c + 3)
        transpose_chunk(c)
        fire_writes(c)
        return carry

    def tail_body(c, carry):
        drain_gather(c)
        drain_writes(c - _R)
        transpose_chunk(c)
        fire_writes(c)
        return carry

    def final_body(c, carry):
        drain_writes(c)
        return carry

    lax.fori_loop(0, _R, warm_body, 0)
    lax.fori_loop(_R, _N_CHUNKS - 3, steady_body, 0)
    lax.fori_loop(_N_CHUNKS - 3, _N_CHUNKS, tail_body, 0)
    lax.fori_loop(_N_CHUNKS - _R, _N_CHUNKS, final_body, 0)


def kernel(tokens, table):
    out4 = _gather_kernel(tokens, table)
    return out4.transpose(1, 3, 0, 2).reshape(N_TOKENS, EMBED_DIM)


# speedup vs baseline: 1.9948x; 1.9948x over previous
import functools

import jax
import jax.numpy as jnp
from jax import lax
from jax.experimental import pallas as pl
from jax.experimental.pallas import tpu as pltpu
from jax.experimental.pallas import tpu_sc as plsc

NUM_EMBEDDINGS = 1000000
EMBED_DIM = 64
N_TOKENS = 819200

_NC = 2
_NS = 16
_NW = _NC * _NS

_CHUNK = 128
_B_PER_W = N_TOKENS // _NW
_N_CHUNKS = _B_PER_W // _CHUNK
_R = 4
_RB = EMBED_DIM // 8
_N_CB = N_TOKENS // _CHUNK

_mesh = plsc.VectorSubcoreMesh(core_axis_name="c", subcore_axis_name="s")


@functools.partial(
    pl.kernel,
    out_type=jax.ShapeDtypeStruct((_RB, _N_CB, 8, _CHUNK), jnp.float32),
    mesh=_mesh,
    scratch_types=[
        pltpu.VMEM((_B_PER_W,), jnp.int32),
        pltpu.VMEM((_R, _CHUNK, EMBED_DIM), jnp.float32),
        pltpu.VMEM((_R, _RB, 8, _CHUNK + 1), jnp.float32),
        pltpu.SemaphoreType.DMA,
        pltpu.SemaphoreType.DMA,
    ],
    compiler_params=pltpu.CompilerParams(
        use_tc_tiling_on_sc=False, needs_layout_passes=False),
)
def _gather_kernel(tokens_hbm, table_hbm, out_hbm, idx_v, rows_v, tiles_v,
                   gsem, wsem):
    wid = lax.axis_index("s") * _NC + lax.axis_index("c")
    base = wid * _B_PER_W
    cb0 = wid * _N_CHUNKS
    pltpu.sync_copy(tokens_hbm.at[pl.ds(base, _B_PER_W)], idx_v)

    lane = lax.iota(jnp.int32, 16)
    rb_idx = [(16 * cg + lane) // 8 for cg in range(EMBED_DIM // 16)]
    sl_idx = lane % 8
    ln_idx = [jnp.full((16,), ln, jnp.int32) for ln in range(_CHUNK)]

    def fire_gather(c):
        pltpu.async_copy(
            table_hbm.at[idx_v.at[pl.ds(c * _CHUNK, _CHUNK)]],
            rows_v.at[c % _R], gsem)

    def drain_gather(c):
        pltpu.make_async_copy(
            table_hbm.at[idx_v.at[pl.ds(c * _CHUNK, _CHUNK)]],
            rows_v.at[c % _R], gsem).wait()

    def transpose_chunk(c):
        slot = c % _R
        tiles = tiles_v.at[slot]
        for ln in range(_CHUNK):
            for cg in range(EMBED_DIM // 16):
                v = rows_v[slot, ln, pl.ds(16 * cg, 16)]
                plsc.store_scatter(tiles, [rb_idx[cg], sl_idx, ln_idx[ln]], v)

    def fire_writes(c):
        for rb in range(_RB):
            pltpu.async_copy(
                tiles_v.at[c % _R, rb, :, pl.ds(0, _CHUNK)],
                out_hbm.at[rb, cb0 + c], wsem)

    def drain_writes(c):
        for rb in range(_RB):
            pltpu.make_async_copy(
                tiles_v.at[c % _R, rb, :, pl.ds(0, _CHUNK)],
                out_hbm.at[rb, cb0 + c], wsem).wait()

    for c in range(3):
        fire_gather(c)

    def warm_body(c, carry):
        drain_gather(c)
        fire_gather(c + 3)
        transpose_chunk(c)
        fire_writes(c)
        return carry

    def steady_body(c, carry):
        drain_gather(c)
        drain_writes(c - _R)
        fire_gather(c + 3)
        transpose_chunk(c)
        fire_writes(c)
        return carry

    def tail_body(c, carry):
        drain_gather(c)
        drain_writes(c - _R)
        transpose_chunk(c)
        fire_writes(c)
        return carry

    def final_body(c, carry):
        drain_writes(c)
        return carry

    lax.fori_loop(0, _R, warm_body, 0)
    lax.fori_loop(_R, _N_CHUNKS - 3, steady_body, 0)
    lax.fori_loop(_N_CHUNKS - 3, _N_CHUNKS, tail_body, 0)
    lax.fori_loop(_N_CHUNKS - _R, _N_CHUNKS, final_body, 0)


def kernel(tokens, table):
    out4 = _gather_kernel(tokens, table)
    return out4.transpose(1, 3, 0, 2).reshape(N_TOKENS, EMBED_DIM)


# speedup vs baseline: 1.9973x; 1.0012x over previous
import functools

import jax
import jax.numpy as jnp
from jax import lax
from jax.experimental import pallas as pl
from jax.experimental.pallas import tpu as pltpu
from jax.experimental.pallas import tpu_sc as plsc

NUM_EMBEDDINGS = 1000000
EMBED_DIM = 64
N_TOKENS = 819200

_NC = 2
_NS = 16
_NW = _NC * _NS

_CHUNK = 128
_B_PER_W = N_TOKENS // _NW
_N_CHUNKS = _B_PER_W // _CHUNK
_R = 4
_AHEAD = _R - 1
_RB = EMBED_DIM // 8
_N_CB = N_TOKENS // _CHUNK

_mesh = plsc.VectorSubcoreMesh(core_axis_name="c", subcore_axis_name="s")


@functools.partial(
    pl.kernel,
    out_type=jax.ShapeDtypeStruct((_RB, _N_CB, 8, _CHUNK), jnp.float32),
    mesh=_mesh,
    scratch_types=[
        pltpu.VMEM((_B_PER_W,), jnp.int32),
        pltpu.VMEM((_R, _CHUNK, EMBED_DIM), jnp.float32),
        pltpu.VMEM((_R, _RB, 8, _CHUNK + 1), jnp.float32),
        pltpu.SemaphoreType.DMA,
        pltpu.SemaphoreType.DMA,
    ],
    compiler_params=pltpu.CompilerParams(
        use_tc_tiling_on_sc=False, needs_layout_passes=False),
)
def _gather_kernel(tokens_hbm, table_hbm, out_hbm, idx_v, rows_v, tiles_v,
                   gsem, wsem):
    wid = lax.axis_index("s") * _NC + lax.axis_index("c")
    base = wid * _B_PER_W
    cb0 = wid * _N_CHUNKS
    pltpu.sync_copy(tokens_hbm.at[pl.ds(base, _B_PER_W)], idx_v)

    lane = lax.iota(jnp.int32, 16)
    rb_idx = [(16 * cg + lane) // 8 for cg in range(EMBED_DIM // 16)]
    sl_idx = lane % 8
    ln_idx = [jnp.full((16,), ln, jnp.int32) for ln in range(_CHUNK)]

    def fire_gather(c):
        pltpu.async_copy(
            table_hbm.at[idx_v.at[pl.ds(c * _CHUNK, _CHUNK)]],
            rows_v.at[c % _R], gsem)

    def drain_gather(c):
        pltpu.make_async_copy(
            table_hbm.at[idx_v.at[pl.ds(c * _CHUNK, _CHUNK)]],
            rows_v.at[c % _R], gsem).wait()

    def transpose_chunk(c):
        slot = c % _R
        tiles = tiles_v.at[slot]
        for ln in range(_CHUNK):
            for cg in range(EMBED_DIM // 16):
                v = rows_v[slot, ln, pl.ds(16 * cg, 16)]
                plsc.store_scatter(tiles, [rb_idx[cg], sl_idx, ln_idx[ln]], v)

    def fire_writes(c):
        for rb in range(_RB):
            pltpu.async_copy(
                tiles_v.at[c % _R, rb, :, pl.ds(0, _CHUNK)],
                out_hbm.at[rb, cb0 + c], wsem)

    def drain_writes(c):
        for rb in range(_RB):
            pltpu.make_async_copy(
                tiles_v.at[c % _R, rb, :, pl.ds(0, _CHUNK)],
                out_hbm.at[rb, cb0 + c], wsem).wait()

    for c in range(_AHEAD):
        fire_gather(c)

    def warm_body(c, carry):
        drain_gather(c)
        fire_gather(c + _AHEAD)
        transpose_chunk(c)
        fire_writes(c)
        return carry

    def steady_body(c, carry):
        drain_gather(c)
        drain_writes(c - _R)
        fire_gather(c + _AHEAD)
        transpose_chunk(c)
        fire_writes(c)
        return carry

    def tail_body(c, carry):
        drain_gather(c)
        drain_writes(c - _R)
        transpose_chunk(c)
        fire_writes(c)
        return carry

    def final_body(c, carry):
        drain_writes(c)
        return carry

    lax.fori_loop(0, _R, warm_body, 0)
    lax.fori_loop(_R, _N_CHUNKS - _AHEAD, steady_body, 0)
    lax.fori_loop(_N_CHUNKS - _AHEAD, _N_CHUNKS, tail_body, 0)
    lax.fori_loop(_N_CHUNKS - _R, _N_CHUNKS, final_body, 0)


def kernel(tokens, table):
    out4 = _gather_kernel(tokens, table)
    return out4.transpose(1, 3, 0, 2).reshape(N_TOKENS, EMBED_DIM)
